# baseline (speedup 1.0000x reference)
import functools

import jax
import jax.numpy as jnp
from jax import lax
from jax.experimental import pallas as pl
from jax.experimental.pallas import tpu as pltpu
from jax.experimental.pallas import tpu_sc as plsc

_B = 1024
_L = 200
_H = 64
_NW = 32
_SEQ_PER_W = _B // _NW
_OFFS = (0, 96)


def _rsqrt_newton(v):
    i = lax.bitcast_convert_type(v, jnp.int32)
    i = jnp.int32(0x5F3759DF) - lax.shift_right_logical(i, 1)
    y = lax.bitcast_convert_type(i, jnp.float32)
    vh = 0.5 * v
    for _ in range(2):
        y = y * (1.5 - vh * y * y)
    return y


def _sc_body(ids_hbm, ids2_hbm, tt_hbm, word_hbm, pos_hbm, type_hbm,
             gamma_hbm, beta_hbm, out_hbm, idx0, idx1, ic0, ic1, tt0, tt1,
             rows0, rows1, ob0, ob1, comb_v, tv_v, g_v, b_v, gsem0, gsem1,
             wsem0, wsem1):
    wid = lax.axis_index("s") * 2 + lax.axis_index("c")
    seq0 = wid * _SEQ_PER_W

    pltpu.sync_copy(pos_hbm.at[pl.ds(0, _L)], comb_v.at[pl.ds(0, _L)])
    pltpu.sync_copy(pos_hbm.at[pl.ds(0, _L)], comb_v.at[pl.ds(_L, _L)])
    pltpu.sync_copy(type_hbm, tv_v)
    pltpu.sync_copy(gamma_hbm, g_v)
    pltpu.sync_copy(beta_hbm, b_v)

    sl = [pl.ds(16 * j, 16) for j in range(4)]
    t0 = [tv_v[0, sl[j]] for j in range(4)]
    t1 = [tv_v[1, sl[j]] for j in range(4)]
    gam = [g_v[sl[j]] for j in range(4)]
    bet = [b_v[sl[j]] for j in range(4)]

    @pl.loop(0, _L)
    def _build(r):
        for j in range(4):
            comb_v[r, sl[j]] = comb_v[r, sl[j]] + t0[j]
            comb_v[_L + r, sl[j]] = comb_v[_L + r, sl[j]] + t1[j]

    bufs = ((idx0, ic0, tt0, rows0, ob0, gsem0, wsem0),
            (idx1, ic1, tt1, rows1, ob1, gsem1, wsem1))

    def wb_copy(c, b):
        ob, wsem = bufs[b][4], bufs[b][6]
        return pltpu.make_async_copy(ob, out_hbm.at[seq0 + c], wsem)

    def issue(c, b):
        idx, ic, tt, rows, _, gsem, _ = bufs[b]
        s = seq0 + c
        for k, off in enumerate(_OFFS):
            pltpu.sync_copy(ids2_hbm.at[s, pl.ds(off, 104)], idx.at[k])
        pltpu.sync_copy(ids_hbm.at[s, pl.ds(0, _L)], ic.at[pl.ds(0, _L)])
        pltpu.sync_copy(tt_hbm.at[s, pl.ds(0, _L)], tt.at[pl.ds(0, _L)])
        for k, off in enumerate(_OFFS):
            pltpu.make_async_copy(word_hbm.at[idx.at[k]],
                                  rows.at[pl.ds(off, 104)], gsem).start()

    def process(c, b):
        idx, ic, tt, rows, ob, gsem, _ = bufs[b]
        for k, off in enumerate(_OFFS):
            pltpu.make_async_copy(word_hbm.at[idx.at[k]],
                                  rows.at[pl.ds(off, 104)], gsem).wait()

        @pl.when(c >= 2)
        def _():
            wb_copy(c - 2, b).wait()

        @pl.loop(0, _L // 8)
        def _group(g):
            rbase = g * 8
            ttv = tt[pl.ds(rbase, 16)]
            idv = ic[pl.ds(rbase, 16)]
            for p in range(2):
                xs, stats = [], []
                for i in range(4):
                    r = rbase + 4 * p + i
                    cb = ttv[4 * p + i] * _L + r
                    co = (idv[4 * p + i] & 1) * _H
                    x = [rows[r, pl.ds(co + 16 * j, 16)] + comb_v[cb, sl[j]]
                         for j in range(4)]
                    s = (x[0] + x[1]) + (x[2] + x[3])
                    sq = ((x[0] * x[0] + x[1] * x[1])
                          + (x[2] * x[2] + x[3] * x[3]))
                    xs.append(x)
                    stats.append((jnp.sum(s), jnp.sum(sq)))
                for i in range(4):
                    r = rbase + 4 * p + i
                    tot, tsq = stats[i]
                    x = xs[i]
                    mean = tot * jnp.float32(1.0 / _H)
                    var = tsq * jnp.float32(1.0 / _H) - mean * mean
                    vv = jnp.full((16,), var + jnp.float32(1e-5), jnp.float32)
                    scale = _rsqrt_newton(vv)
                    mv = jnp.full((16,), mean, jnp.float32)
                    for j in range(4):
                        ob[r, sl[j]] = (x[j] - mv) * scale * gam[j] + bet[j]

        wb_copy(c, b).start()

    issue(0, 0)

    @pl.loop(0, _SEQ_PER_W // 2)
    def _main(i):
        c0 = i * 2
        issue(c0 + 1, 1)
        process(c0, 0)

        @pl.when(c0 + 2 < _SEQ_PER_W)
        def _():
            issue(c0 + 2, 0)

        process(c0 + 1, 1)

    wb_copy(_SEQ_PER_W - 2, 0).wait()
    wb_copy(_SEQ_PER_W - 1, 1).wait()


def kernel(input_ids, token_type_ids, word_emb, pos_emb, type_emb, gamma,
           beta):
    ids = input_ids.astype(jnp.int32)
    word2 = word_emb.reshape(word_emb.shape[0] // 2, 2 * _H)

    mesh = plsc.VectorSubcoreMesh(core_axis_name="c", subcore_axis_name="s")
    run = functools.partial(
        pl.kernel,
        mesh=mesh,
        compiler_params=pltpu.CompilerParams(
            needs_layout_passes=False, use_tc_tiling_on_sc=False),
        out_type=jax.ShapeDtypeStruct((_B, _L, _H), jnp.float32),
        scratch_types=[
            pltpu.VMEM((2, 104), jnp.int32),
            pltpu.VMEM((2, 104), jnp.int32),
            pltpu.VMEM((208,), jnp.int32),
            pltpu.VMEM((208,), jnp.int32),
            pltpu.VMEM((208,), jnp.int32),
            pltpu.VMEM((208,), jnp.int32),
            pltpu.VMEM((_L, 2 * _H), jnp.float32),
            pltpu.VMEM((_L, 2 * _H), jnp.float32),
            pltpu.VMEM((_L, _H), jnp.float32),
            pltpu.VMEM((_L, _H), jnp.float32),
            pltpu.VMEM((2 * _L, _H), jnp.float32),
            pltpu.VMEM((2, _H), jnp.float32),
            pltpu.VMEM((_H,), jnp.float32),
            pltpu.VMEM((_H,), jnp.float32),
            pltpu.SemaphoreType.DMA,
            pltpu.SemaphoreType.DMA,
            pltpu.SemaphoreType.DMA,
            pltpu.SemaphoreType.DMA,
        ],
    )(_sc_body)
    return run(ids, ids >> 1, token_type_ids.astype(jnp.int32), word2,
               pos_emb, type_emb, gamma, beta)


# speedup vs baseline: 1.1747x; 1.1747x over previous
import functools

import jax
import jax.numpy as jnp
from jax import lax
from jax.experimental import pallas as pl
from jax.experimental.pallas import tpu as pltpu
from jax.experimental.pallas import tpu_sc as plsc

_B = 1024
_L = 200
_H = 64
_NW = 32
_SEQ_PER_W = _B // _NW
_OFFS = (0, 96)


def _rsqrt_newton(v):
    i = lax.bitcast_convert_type(v, jnp.int32)
    i = jnp.int32(0x5F3759DF) - lax.shift_right_logical(i, 1)
    y = lax.bitcast_convert_type(i, jnp.float32)
    vh = 0.5 * v
    for _ in range(2):
        y = y * (1.5 - vh * y * y)
    return y


def _sc_body(ids_hbm, ids2_hbm, tt_hbm, word_hbm, pos_hbm, type_hbm,
             gamma_hbm, beta_hbm, out_hbm, idx0, idx1, ic0, ic1, tt0, tt1,
             rows0, rows1, ob0, ob1, comb_v, tv_v, g_v, b_v, gsem0, gsem1,
             wsem0, wsem1):
    wid = lax.axis_index("s") * 2 + lax.axis_index("c")
    seq0 = wid * _SEQ_PER_W

    pltpu.sync_copy(pos_hbm.at[pl.ds(0, _L)], comb_v.at[pl.ds(0, _L)])
    pltpu.sync_copy(pos_hbm.at[pl.ds(0, _L)], comb_v.at[pl.ds(_L, _L)])
    pltpu.sync_copy(type_hbm, tv_v)
    pltpu.sync_copy(gamma_hbm, g_v)
    pltpu.sync_copy(beta_hbm, b_v)

    sl = [pl.ds(16 * j, 16) for j in range(4)]
    t0 = [tv_v[0, sl[j]] for j in range(4)]
    t1 = [tv_v[1, sl[j]] for j in range(4)]
    gam = [g_v[sl[j]] for j in range(4)]
    bet = [b_v[sl[j]] for j in range(4)]

    @pl.loop(0, _L)
    def _build(r):
        for j in range(4):
            comb_v[r, sl[j]] = comb_v[r, sl[j]] + t0[j]
            comb_v[_L + r, sl[j]] = comb_v[_L + r, sl[j]] + t1[j]

    bufs = ((idx0, ic0, tt0, rows0, ob0, gsem0, wsem0),
            (idx1, ic1, tt1, rows1, ob1, gsem1, wsem1))

    def wb_copy(c, b):
        ob, wsem = bufs[b][4], bufs[b][6]
        return pltpu.make_async_copy(ob, out_hbm.at[seq0 + c], wsem)

    def issue(c, b):
        idx, ic, tt, rows, _, gsem, _ = bufs[b]
        s = seq0 + c
        for k, off in enumerate(_OFFS):
            pltpu.sync_copy(ids_hbm.at[s, pl.ds(off, 104)], idx.at[k])
        pltpu.sync_copy(tt_hbm.at[s, pl.ds(0, _L)], tt.at[pl.ds(0, _L)])
        for k, off in enumerate(_OFFS):
            pltpu.make_async_copy(word_hbm.at[idx.at[k]],
                                  rows.at[pl.ds(off, 104)], gsem).start()

    def process(c, b):
        idx, ic, tt, rows, ob, gsem, _ = bufs[b]
        for k, off in enumerate(_OFFS):
            pltpu.make_async_copy(word_hbm.at[idx.at[k]],
                                  rows.at[pl.ds(off, 104)], gsem).wait()

        @pl.when(c >= 2)
        def _():
            wb_copy(c - 2, b).wait()

        @pl.loop(0, _L // 8)
        def _group(g):
            rbase = g * 8
            ttv = tt[pl.ds(rbase, 16)]
            for p in range(2):
                xs, stats = [], []
                for i in range(4):
                    r = rbase + 4 * p + i
                    cb = ttv[4 * p + i] * _L + r
                    x = [rows[r, sl[j]] + comb_v[cb, sl[j]]
                         for j in range(4)]
                    s = (x[0] + x[1]) + (x[2] + x[3])
                    sq = ((x[0] * x[0] + x[1] * x[1])
                          + (x[2] * x[2] + x[3] * x[3]))
                    xs.append(x)
                    stats.append((jnp.sum(s), jnp.sum(sq)))
                for i in range(4):
                    r = rbase + 4 * p + i
                    tot, tsq = stats[i]
                    x = xs[i]
                    mean = tot * jnp.float32(1.0 / _H)
                    var = tsq * jnp.float32(1.0 / _H) - mean * mean
                    vv = jnp.full((16,), var + jnp.float32(1e-5), jnp.float32)
                    scale = _rsqrt_newton(vv)
                    mv = jnp.full((16,), mean, jnp.float32)
                    for j in range(4):
                        ob[r, sl[j]] = (x[j] - mv) * scale * gam[j] + bet[j]

        wb_copy(c, b).start()

    issue(0, 0)

    @pl.loop(0, _SEQ_PER_W // 2)
    def _main(i):
        c0 = i * 2
        issue(c0 + 1, 1)
        process(c0, 0)

        @pl.when(c0 + 2 < _SEQ_PER_W)
        def _():
            issue(c0 + 2, 0)

        process(c0 + 1, 1)

    wb_copy(_SEQ_PER_W - 2, 0).wait()
    wb_copy(_SEQ_PER_W - 1, 1).wait()


def kernel(input_ids, token_type_ids, word_emb, pos_emb, type_emb, gamma,
           beta):
    ids = input_ids.astype(jnp.int32)
    wordp = jnp.pad(word_emb, ((0, 0), (0, _H)))

    mesh = plsc.VectorSubcoreMesh(core_axis_name="c", subcore_axis_name="s")
    run = functools.partial(
        pl.kernel,
        mesh=mesh,
        compiler_params=pltpu.CompilerParams(
            needs_layout_passes=False, use_tc_tiling_on_sc=False),
        out_type=jax.ShapeDtypeStruct((_B, _L, _H), jnp.float32),
        scratch_types=[
            pltpu.VMEM((2, 104), jnp.int32),
            pltpu.VMEM((2, 104), jnp.int32),
            pltpu.VMEM((208,), jnp.int32),
            pltpu.VMEM((208,), jnp.int32),
            pltpu.VMEM((208,), jnp.int32),
            pltpu.VMEM((208,), jnp.int32),
            pltpu.VMEM((_L, 2 * _H), jnp.float32),
            pltpu.VMEM((_L, 2 * _H), jnp.float32),
            pltpu.VMEM((_L, _H), jnp.float32),
            pltpu.VMEM((_L, _H), jnp.float32),
            pltpu.VMEM((2 * _L, _H), jnp.float32),
            pltpu.VMEM((2, _H), jnp.float32),
            pltpu.VMEM((_H,), jnp.float32),
            pltpu.VMEM((_H,), jnp.float32),
            pltpu.SemaphoreType.DMA,
            pltpu.SemaphoreType.DMA,
            pltpu.SemaphoreType.DMA,
            pltpu.SemaphoreType.DMA,
        ],
    )(_sc_body)
    return run(ids, ids, token_type_ids.astype(jnp.int32), wordp,
               pos_emb, type_emb, gamma, beta)


# speedup vs baseline: 1.2375x; 1.0534x over previous
import functools

import jax
import jax.numpy as jnp
from jax import lax
from jax.experimental import pallas as pl
from jax.experimental.pallas import tpu as pltpu
from jax.experimental.pallas import tpu_sc as plsc

_B = 1024
_L = 200
_H = 64
_NW = 32
_SEQ_PER_W = _B // _NW
_OFFS = (0, 96)
_LH = _L * _H


def _rsqrt_newton(v):
    i = lax.bitcast_convert_type(v, jnp.int32)
    i = jnp.int32(0x5F3759DF) - lax.shift_right_logical(i, 1)
    y = lax.bitcast_convert_type(i, jnp.float32)
    vh = 0.5 * v
    for _ in range(2):
        y = y * (1.5 - vh * y * y)
    return y


def _sc_body(ids_hbm, tt_hbm, word_hbm, pos_hbm, type_hbm, gamma_hbm,
             beta_hbm, out_hbm, idx0, idx1, tt0, tt1, rows0, rows1, ob0, ob1,
             comb_v, tv_v, g_v, b_v, ssem0, ssem1, gsem0, gsem1, wsem0,
             wsem1):
    wid = lax.axis_index("s") * 2 + lax.axis_index("c")
    seq0 = wid * _SEQ_PER_W

    pltpu.sync_copy(pos_hbm.at[pl.ds(0, _L)], comb_v.at[pl.ds(0, _L)])
    pltpu.sync_copy(pos_hbm.at[pl.ds(0, _L)], comb_v.at[pl.ds(_L, _L)])
    pltpu.sync_copy(type_hbm, tv_v)
    pltpu.sync_copy(gamma_hbm, g_v)
    pltpu.sync_copy(beta_hbm, b_v)

    sl = [pl.ds(16 * j, 16) for j in range(4)]
    t0 = [tv_v[0, sl[j]] for j in range(4)]
    t1 = [tv_v[1, sl[j]] for j in range(4)]
    gam = [g_v[sl[j]] for j in range(4)]
    bet = [b_v[sl[j]] for j in range(4)]

    @pl.loop(0, _L)
    def _build(r):
        for j in range(4):
            comb_v[r, sl[j]] = comb_v[r, sl[j]] + t0[j]
            comb_v[_L + r, sl[j]] = comb_v[_L + r, sl[j]] + t1[j]

    bufs = ((idx0, tt0, rows0, ob0, ssem0, gsem0, wsem0),
            (idx1, tt1, rows1, ob1, ssem1, gsem1, wsem1))

    def stage_copies(c, b):
        idx, tt, _, _, ssem, _, _ = bufs[b]
        s = seq0 + c
        cps = [pltpu.make_async_copy(ids_hbm.at[s, pl.ds(off, 104)],
                                     idx.at[k], ssem)
               for k, off in enumerate(_OFFS)]
        cps.append(pltpu.make_async_copy(tt_hbm.at[s, pl.ds(0, _L)],
                                         tt.at[pl.ds(0, _L)], ssem))
        return cps

    def gather_copies(b):
        idx, _, rows, _, _, gsem, _ = bufs[b]
        return [pltpu.make_async_copy(word_hbm.at[idx.at[k]],
                                      rows.at[pl.ds(off, 104)], gsem)
                for k, off in enumerate(_OFFS)]

    def stage(c, b):
        for cp in stage_copies(c, b):
            cp.start()

    def fire(c, b):
        for cp in stage_copies(c, b):
            cp.wait()
        for cp in gather_copies(b):
            cp.start()

    def wb_copy(c, b):
        ob, wsem = bufs[b][3], bufs[b][6]
        return pltpu.make_async_copy(
            ob, out_hbm.at[pl.ds((seq0 + c) * _LH, _LH)], wsem)

    def process(c, b):
        _, tt, rows, ob, _, _, _ = bufs[b]
        for cp in gather_copies(b):
            cp.wait()

        @pl.when(c + 2 < _SEQ_PER_W)
        def _():
            stage(c + 2, b)

        @pl.when(c >= 2)
        def _():
            wb_copy(c - 2, b).wait()

        @pl.loop(0, _L // 8)
        def _group(g):
            rbase = g * 8
            ttv = tt[pl.ds(rbase, 16)]
            for p in range(2):
                xs, stats = [], []
                for i in range(4):
                    r = rbase + 4 * p + i
                    cb = ttv[4 * p + i] * _L + r
                    x = [rows[r, sl[j]] + comb_v[cb, sl[j]]
                         for j in range(4)]
                    s = (x[0] + x[1]) + (x[2] + x[3])
                    sq = ((x[0] * x[0] + x[1] * x[1])
                          + (x[2] * x[2] + x[3] * x[3]))
                    xs.append(x)
                    stats.append((jnp.sum(s), jnp.sum(sq)))
                for i in range(4):
                    r = rbase + 4 * p + i
                    tot, tsq = stats[i]
                    x = xs[i]
                    mean = tot * jnp.float32(1.0 / _H)
                    var = tsq * jnp.float32(1.0 / _H) - mean * mean
                    vv = jnp.full((16,), var + jnp.float32(1e-5), jnp.float32)
                    scale = _rsqrt_newton(vv)
                    mv = jnp.full((16,), mean, jnp.float32)
                    for j in range(4):
                        ob[pl.ds(r * _H + 16 * j, 16)] = (
                            (x[j] - mv) * scale * gam[j] + bet[j])

        wb_copy(c, b).start()

    stage(0, 0)
    fire(0, 0)
    stage(1, 1)
    fire(1, 1)

    @pl.loop(0, _SEQ_PER_W // 2)
    def _main(i):
        c0 = i * 2
        process(c0, 0)

        @pl.when(c0 + 2 < _SEQ_PER_W)
        def _():
            fire(c0 + 2, 0)

        process(c0 + 1, 1)

        @pl.when(c0 + 3 < _SEQ_PER_W)
        def _():
            fire(c0 + 3, 1)

    wb_copy(_SEQ_PER_W - 2, 0).wait()
    wb_copy(_SEQ_PER_W - 1, 1).wait()


def kernel(input_ids, token_type_ids, word_emb, pos_emb, type_emb, gamma,
           beta):
    ids = input_ids.astype(jnp.int32)
    wordp = jnp.pad(word_emb, ((0, 0), (0, _H)))

    mesh = plsc.VectorSubcoreMesh(core_axis_name="c", subcore_axis_name="s")
    run = functools.partial(
        pl.kernel,
        mesh=mesh,
        compiler_params=pltpu.CompilerParams(
            needs_layout_passes=False, use_tc_tiling_on_sc=False),
        out_type=jax.ShapeDtypeStruct((_B * _L * _H,), jnp.float32),
        scratch_types=[
            pltpu.VMEM((2, 104), jnp.int32),
            pltpu.VMEM((2, 104), jnp.int32),
            pltpu.VMEM((208,), jnp.int32),
            pltpu.VMEM((208,), jnp.int32),
            pltpu.VMEM((_L, 2 * _H), jnp.float32),
            pltpu.VMEM((_L, 2 * _H), jnp.float32),
            pltpu.VMEM((_LH,), jnp.float32),
            pltpu.VMEM((_LH,), jnp.float32),
            pltpu.VMEM((2 * _L, _H), jnp.float32),
            pltpu.VMEM((2, _H), jnp.float32),
            pltpu.VMEM((_H,), jnp.float32),
            pltpu.VMEM((_H,), jnp.float32),
            pltpu.SemaphoreType.DMA,
            pltpu.SemaphoreType.DMA,
            pltpu.SemaphoreType.DMA,
            pltpu.SemaphoreType.DMA,
            pltpu.SemaphoreType.DMA,
            pltpu.SemaphoreType.DMA,
        ],
    )(_sc_body)
    out = run(ids, token_type_ids.astype(jnp.int32), wordp, pos_emb,
              type_emb, gamma, beta)
    return out.reshape(_B, _L, _H)
